# baseline (speedup 1.0000x reference)
import functools

import numpy as np
import jax
import jax.numpy as jnp
from jax import lax
from jax.experimental import pallas as pl
from jax.experimental.pallas import tpu as pltpu
from jax.experimental.pallas import tpu_sc as plsc

_N = 10000
_NBUF = 4
_LANES = 16


def _pos_encoding(seq_len: int, d: int) -> np.ndarray:
    k = np.arange(seq_len, dtype=np.float64)[:, None]
    i = np.arange(d // 2, dtype=np.float64)[None, :]
    denom = np.power(float(_N), 2.0 * i / d)
    p = np.zeros((seq_len, d), dtype=np.float32)
    p[:, 0::2] = np.sin(k / denom).astype(np.float32)
    p[:, 1::2] = np.cos(k / denom).astype(np.float32)
    return p


@functools.lru_cache(maxsize=None)
def _build(B: int, S: int, D: int, V: int):
    nc, ns = 2, 16
    nw = nc * ns
    assert B % nw == 0
    spw = B // nw
    assert spw % _NBUF == 0
    s_lo = min(S, 128)
    s_hi = S - s_lo

    mesh = plsc.VectorSubcoreMesh(
        core_axis_name="c", subcore_axis_name="s", num_cores=nc, num_subcores=ns)

    @functools.partial(
        pl.kernel,
        out_type=jax.ShapeDtypeStruct((B * S, D), jnp.float32),
        mesh=mesh,
        compiler_params=pltpu.CompilerParams(use_tc_tiling_on_sc=False),
        scratch_types=[
            pltpu.VMEM((spw * S,), jnp.int32),
            pltpu.VMEM((S, D), jnp.float32),
            [pltpu.VMEM((S, D), jnp.float32) for _ in range(_NBUF)],
            [pltpu.SemaphoreType.DMA for _ in range(_NBUF)],
            [pltpu.SemaphoreType.DMA for _ in range(_NBUF)],
        ],
    )
    def fn(x_hbm, pe_hbm, table_hbm, out_hbm, idx_v, pe_v, bufs, gsems, ssems):
        wid = lax.axis_index("s") * nc + lax.axis_index("c")
        seq0 = wid * spw
        pltpu.sync_copy(x_hbm.at[pl.ds(seq0 * S, spw * S)], idx_v)
        pltpu.sync_copy(pe_hbm, pe_v)

        def gather_starts(j, b):
            offs = j * S
            d0 = pltpu.async_copy(
                table_hbm.at[idx_v.at[pl.ds(offs, s_lo)]],
                bufs[b].at[pl.ds(0, s_lo)], gsems[b])
            descs = [d0]
            if s_hi:
                descs.append(pltpu.async_copy(
                    table_hbm.at[idx_v.at[pl.ds(offs + s_lo, s_hi)]],
                    bufs[b].at[pl.ds(s_lo, s_hi)], gsems[b]))
            return descs

        def scatter_wait(b):
            pltpu.make_async_copy(
                bufs[b], out_hbm.at[pl.ds(seq0 * S, S)], ssems[b]).wait()

        def add_pe(b):
            buf = bufs[b]

            @pl.loop(0, S, unroll=8)
            def _(i):
                for k2 in range(D // _LANES):
                    sl = pl.ds(k2 * _LANES, _LANES)
                    plsc.addupdate(buf.at[i, sl], pe_v[i, sl])

        @pl.loop(0, spw, step=_NBUF)
        def _(jj):
            descs = []
            for b in range(_NBUF):
                @pl.when(jj > 0)
                def _():
                    scatter_wait(b)
                descs.append(gather_starts(jj + b, b))
            for b in range(_NBUF):
                for d in descs[b]:
                    d.wait()
                add_pe(b)
                pltpu.async_copy(
                    bufs[b],
                    out_hbm.at[pl.ds((seq0 + jj + b) * S, S)],
                    ssems[b])

        for b in range(_NBUF):
            scatter_wait(b)

    return fn


def kernel(x, table):
    B, S = x.shape
    V, D = table.shape
    pe = jnp.asarray(_pos_encoding(S, D))
    fn = _build(B, S, D, V)
    out = fn(x.reshape(B * S), pe, table)
    return out.reshape(B, S, D)
